# baseline (speedup 1.0000x reference)
import math

import jax
import jax.numpy as jnp
from jax import lax
from jax.experimental import pallas as pl
from jax.experimental.pallas import tpu as pltpu

N = 2048
H = 16
NBUCKETS = 32
MAX_DISTANCE = 128
NV = 16
NS = 8
LW = 4480
LEXT = 4736

_ROWS_PER_BLOCK = 16
_PANELS_PER_BLOCK = _ROWS_PER_BLOCK // NS
_GRID = N // _ROWS_PER_BLOCK


def _body(table_t_ref, out_ref, skew_ref, acc_ref):
    g = pl.program_id(0)

    @pl.when(g == 0)
    def _build_skew():
        u = lax.broadcasted_iota(jnp.int32, (1, LEXT), 1)
        nv_ = (N - 1) - u
        neg = jnp.where(nv_ < 0, NBUCKETS // 2, 0)
        a = jnp.abs(nv_)
        small = a < (NBUCKETS // 4)
        safe = jnp.maximum(a, 1).astype(jnp.float32)
        t = jnp.log(safe / (NBUCKETS // 4))
        t = t / math.log(MAX_DISTANCE / (NBUCKETS // 4))
        t = t * (NBUCKETS // 2 - NBUCKETS // 4)
        large = (NBUCKETS // 4) + t.astype(jnp.int32)
        large = jnp.minimum(large, NBUCKETS // 2 - 1)
        bucket = neg + jnp.where(small, a, large)

        acc = jnp.zeros((H, LEXT), jnp.float32)
        for b in range(NBUCKETS):
            m = (bucket == b).astype(jnp.float32)
            acc = acc + table_t_ref[:, b:b + 1] * m
        acc_ref[...] = acc
        for v in range(NV):
            for s in range(NS):
                off = 8 * v + 7 - s
                skew_ref[v, :, s, :] = acc_ref[:, off:off + LW]

    for p in range(_PANELS_PER_BLOCK):
        it = g * _PANELS_PER_BLOCK + p
        v = jnp.bitwise_and((N // NS - 1) - it, NV - 1)
        q = pl.multiple_of(2040 - 8 * it - 8 * v, 128)
        out_ref[:, p * NS:(p + 1) * NS, :] = skew_ref[v, :, :, pl.ds(q, N)]


def kernel(n, relative_attention_bias):
    del n
    table_t = relative_attention_bias.T.astype(jnp.float32)
    return pl.pallas_call(
        _body,
        grid=(_GRID,),
        in_specs=[pl.BlockSpec((H, NBUCKETS), lambda g: (0, 0))],
        out_specs=pl.BlockSpec((H, _ROWS_PER_BLOCK, N), lambda g: (0, g, 0)),
        out_shape=jax.ShapeDtypeStruct((H, N, N), jnp.float32),
        scratch_shapes=[
            pltpu.VMEM((NV, H, NS, LW), jnp.float32),
            pltpu.VMEM((H, LEXT), jnp.float32),
        ],
        compiler_params=pltpu.CompilerParams(
            dimension_semantics=("arbitrary",),
            vmem_limit_bytes=100 * 1024 * 1024,
        ),
    )(table_t)


# speedup vs baseline: 1.2230x; 1.2230x over previous
import math

import jax
import jax.numpy as jnp
from jax import lax
from jax.experimental import pallas as pl
from jax.experimental.pallas import tpu as pltpu

N = 2048
H = 16
NBUCKETS = 32
MAX_DISTANCE = 128
NV = 16
NS = 8
LW = 4480
LEXT = 4736

_ROWS_PER_BLOCK = 32
_PANELS_PER_BLOCK = _ROWS_PER_BLOCK // NS
_GRID = N // _ROWS_PER_BLOCK


def _body(table_t_ref, out_ref, skew_ref, acc_ref):
    g = pl.program_id(0)

    @pl.when(g == 0)
    def _build_skew():
        u = lax.broadcasted_iota(jnp.int32, (1, LEXT), 1)
        nv_ = (N - 1) - u
        neg = jnp.where(nv_ < 0, NBUCKETS // 2, 0)
        a = jnp.abs(nv_)
        small = a < (NBUCKETS // 4)
        safe = jnp.maximum(a, 1).astype(jnp.float32)
        t = jnp.log(safe / (NBUCKETS // 4))
        t = t / math.log(MAX_DISTANCE / (NBUCKETS // 4))
        t = t * (NBUCKETS // 2 - NBUCKETS // 4)
        large = (NBUCKETS // 4) + t.astype(jnp.int32)
        large = jnp.minimum(large, NBUCKETS // 2 - 1)
        bucket = neg + jnp.where(small, a, large)

        acc = jnp.zeros((H, LEXT), jnp.float32)
        for b in range(NBUCKETS):
            m = (bucket == b).astype(jnp.float32)
            acc = acc + table_t_ref[:, b:b + 1] * m
        acc_ref[...] = acc

    for gg in range(4):
        @pl.when(g == gg)
        def _build_skew_part(gg=gg):
            for v in range(12 - 4 * gg, 16 - 4 * gg):
                for s in range(NS):
                    off = 8 * v + 7 - s
                    skew_ref[v, :, s, :] = acc_ref[:, off:off + LW]

    for p in range(_PANELS_PER_BLOCK):
        it = g * _PANELS_PER_BLOCK + p
        v = jnp.bitwise_and((N // NS - 1) - it, NV - 1)
        q = pl.multiple_of(2040 - 8 * it - 8 * v, 128)
        out_ref[:, p * NS:(p + 1) * NS, :] = skew_ref[v, :, :, pl.ds(q, N)]


def kernel(n, relative_attention_bias):
    del n
    table_t = relative_attention_bias.T.astype(jnp.float32)
    return pl.pallas_call(
        _body,
        grid=(_GRID,),
        in_specs=[pl.BlockSpec((H, NBUCKETS), lambda g: (0, 0))],
        out_specs=pl.BlockSpec((H, _ROWS_PER_BLOCK, N), lambda g: (0, g, 0)),
        out_shape=jax.ShapeDtypeStruct((H, N, N), jnp.float32),
        scratch_shapes=[
            pltpu.VMEM((NV, H, NS, LW), jnp.float32),
            pltpu.VMEM((H, LEXT), jnp.float32),
        ],
        compiler_params=pltpu.CompilerParams(
            dimension_semantics=("arbitrary",),
            vmem_limit_bytes=100 * 1024 * 1024,
        ),
    )(table_t)


# speedup vs baseline: 1.2290x; 1.0049x over previous
import math

import jax
import jax.numpy as jnp
from jax import lax
from jax.experimental import pallas as pl
from jax.experimental.pallas import tpu as pltpu

N = 2048
H = 16
NBUCKETS = 32
MAX_DISTANCE = 128
NV = 4
NS = 32
LW = 4480
LEXT = 4736

_ROWS_PER_BLOCK = 32
_PANELS_PER_BLOCK = _ROWS_PER_BLOCK // NS
_GRID = N // _ROWS_PER_BLOCK


def _body(table_t_ref, out_ref, skew_ref, acc_ref):
    g = pl.program_id(0)

    @pl.when(g == 0)
    def _build_skew():
        u = lax.broadcasted_iota(jnp.int32, (1, LEXT), 1)
        nv_ = (N - 1) - u
        neg = jnp.where(nv_ < 0, NBUCKETS // 2, 0)
        a = jnp.abs(nv_)
        small = a < (NBUCKETS // 4)
        safe = jnp.maximum(a, 1).astype(jnp.float32)
        t = jnp.log(safe / (NBUCKETS // 4))
        t = t / math.log(MAX_DISTANCE / (NBUCKETS // 4))
        t = t * (NBUCKETS // 2 - NBUCKETS // 4)
        large = (NBUCKETS // 4) + t.astype(jnp.int32)
        large = jnp.minimum(large, NBUCKETS // 2 - 1)
        bucket = neg + jnp.where(small, a, large)

        acc = jnp.zeros((H, LEXT), jnp.float32)
        for b in range(NBUCKETS):
            m = (bucket == b).astype(jnp.float32)
            acc = acc + table_t_ref[:, b:b + 1] * m
        acc_ref[...] = acc

    for gg in range(4):
        @pl.when(g == gg)
        def _build_skew_part(gg=gg):
            v = (3 - gg) & 3
            for s in range(NS):
                off = 32 * v + 31 - s
                skew_ref[v, :, s, :] = acc_ref[:, off:off + LW]

    v = jnp.bitwise_and(3 - g, 3)
    q = pl.multiple_of(2016 - 32 * g - 32 * v, 128)
    out_ref[...] = skew_ref[v, :, :, pl.ds(q, N)]


def kernel(n, relative_attention_bias):
    del n
    table_t = relative_attention_bias.T.astype(jnp.float32)
    return pl.pallas_call(
        _body,
        grid=(_GRID,),
        in_specs=[pl.BlockSpec((H, NBUCKETS), lambda g: (0, 0))],
        out_specs=pl.BlockSpec((H, _ROWS_PER_BLOCK, N), lambda g: (0, g, 0)),
        out_shape=jax.ShapeDtypeStruct((H, N, N), jnp.float32),
        scratch_shapes=[
            pltpu.VMEM((NV, H, NS, LW), jnp.float32),
            pltpu.VMEM((H, LEXT), jnp.float32),
        ],
        compiler_params=pltpu.CompilerParams(
            dimension_semantics=("arbitrary",),
            vmem_limit_bytes=100 * 1024 * 1024,
        ),
    )(table_t)
